# Initial kernel scaffold; baseline (speedup 1.0000x reference)
import jax
import jax.numpy as jnp
from jax.experimental import pallas as pl


def kernel(x, prob, W_fix, W_qkv, W_out, b_out):
    raise NotImplementedError("write your pallas kernel here")


# speedup vs baseline: 47.5386x; 47.5386x over previous
import numpy as np
import jax
import jax.numpy as jnp
from jax.experimental import pallas as pl
from jax.experimental.pallas import tpu as pltpu

WIN = 16
HEADS = 8
DIM_HEAD = 64
ATT_SCALE = DIM_HEAD ** -0.5
GRID = 57
KEEP = 44
PAD = 4
H = 128
D = 192
INNER = HEADS * DIM_HEAD


def _interp_matrix():
    M = np.zeros((WIN, WIN), dtype=np.float32)
    scale = 15.0 / WIN
    for i in range(WIN):
        y = (i + 0.5) * scale
        p0 = int(np.floor(y))
        ly = y - p0
        M[i, p0] += 1.0 - ly
        M[i, p0 + 1] += ly
    return M


_A = np.kron(_interp_matrix(), _interp_matrix())


def _nms_offsets():
    offs = []
    for gdy in range(-PAD, PAD + 1):
        for gdx in range(-PAD, PAD + 1):
            if gdy == 0 and gdx == 0:
                continue
            dx, dy = 2.0 * gdx, 2.0 * gdy
            iw = max(0.0, min(15.0, dx + 15.0) - max(0.0, dx))
            ih = max(0.0, min(15.0, dy + 15.0) - max(0.0, dy))
            inter = iw * ih
            if inter / (450.0 - inter) > 0.2:
                tb = (gdy < 0) or (gdy == 0 and gdx < 0)
                offs.append((gdy, gdx, tb))
    return offs


_OFFS = _nms_offsets()


def _body(prob_ref, x_ref, wfix_ref, A_ref, wqkvT_ref, woutT_ref, bout_ref,
          out_ref, coords_ref):
    t = pl.program_id(0)

    @pl.when(t == 0)
    def _select():
        out_ref[:] = jnp.zeros((H, H, D), jnp.float32)

        p = prob_ref[:]
        ent = -jnp.sum(p * jnp.log2(p + 1e-10), axis=0)
        ws = jnp.zeros((GRID, GRID), jnp.float32)
        for u in range(8):
            for v in range(8):
                ws = ws + wfix_ref[u * 8 + v] * jax.lax.slice(
                    ent, (u, v), (u + GRID, v + GRID))
        ws = ws / 64.0

        neg = jnp.float32(-3.4e38)

        def pad2d(a, fill, dtype):
            side = jnp.full((GRID, PAD), fill, dtype)
            mid = jnp.concatenate([side, a, side], axis=1)
            top = jnp.full((PAD, GRID + 2 * PAD), fill, dtype)
            return jnp.concatenate([top, mid, top], axis=0)

        sp = pad2d(ws, neg, jnp.float32)
        higher = []
        for (dy, dx, tb) in _OFFS:
            snb = jax.lax.slice(
                sp, (PAD + dy, PAD + dx), (PAD + dy + GRID, PAD + dx + GRID))
            hi = snb > ws
            if tb:
                hi = hi | (snb == ws)
            higher.append(hi.astype(jnp.float32))

        def one_round(kf):
            kp = pad2d(kf, jnp.float32(0), jnp.float32)
            threat = jnp.zeros((GRID, GRID), jnp.float32)
            for (dy, dx, _), hi in zip(_OFFS, higher):
                knb = jax.lax.slice(
                    kp, (PAD + dy, PAD + dx), (PAD + dy + GRID, PAD + dx + GRID))
                threat = jnp.maximum(threat, knb * hi)
            return 1.0 - threat

        def cond(c):
            return c[1]

        def bd(c):
            kf, _ = c
            nk = one_round(kf)
            return nk, jnp.any(nk != kf)

        keptf, _ = jax.lax.while_loop(
            cond, bd, (jnp.ones((GRID, GRID), jnp.float32), jnp.bool_(True)))
        kept = keptf > 0.5

        iy = jax.lax.broadcasted_iota(jnp.int32, (GRID, GRID), 0)
        ix = jax.lax.broadcasted_iota(jnp.int32, (GRID, GRID), 1)
        lin = iy * GRID + ix
        big = jnp.int32(10 ** 8)

        def pick(j, takenf):
            avail = keptf * (1.0 - takenf)
            sc = jnp.where(avail > 0.5, ws, neg)
            m = jnp.max(sc)
            sl = jnp.min(jnp.where(sc == m, lin, big))
            coords_ref[j, 0] = (sl // GRID) * 2
            coords_ref[j, 1] = (sl % GRID) * 2
            return jnp.maximum(takenf, (lin == sl).astype(jnp.float32))

        jax.lax.fori_loop(0, KEEP, pick, jnp.zeros((GRID, GRID), jnp.float32))

    sy = coords_ref[t, 0]
    sx = coords_ref[t, 1]
    sx8 = jnp.minimum((sx // 8) * 8, H - 24)
    off = sx - sx8
    wide = x_ref[pl.ds(sy, WIN), pl.ds(sx8, 24), :]
    patch = jnp.where(off == 0, wide[:, 0:WIN, :], wide[:, 8:8 + WIN, :])
    for k in (1, 2, 3):
        patch = jnp.where(off == 2 * k, wide[:, 2 * k:2 * k + WIN, :], patch)
    patch = patch.reshape(WIN * WIN, D)
    xi = jnp.dot(A_ref[:], patch, preferred_element_type=jnp.float32)
    qkv = jnp.dot(xi, wqkvT_ref[:], preferred_element_type=jnp.float32)
    outs = []
    for hh in range(HEADS):
        qh = qkv[:, hh * DIM_HEAD:(hh + 1) * DIM_HEAD]
        kh = qkv[:, INNER + hh * DIM_HEAD:INNER + (hh + 1) * DIM_HEAD]
        vh = qkv[:, 2 * INNER + hh * DIM_HEAD:2 * INNER + (hh + 1) * DIM_HEAD]
        dots = jax.lax.dot_general(
            qh, kh, (((1,), (1,)), ((), ())),
            preferred_element_type=jnp.float32) * ATT_SCALE
        mx = jnp.max(dots, axis=1, keepdims=True)
        e = jnp.exp(dots - mx)
        a = e / jnp.sum(e, axis=1, keepdims=True)
        outs.append(jnp.dot(a, vh, preferred_element_type=jnp.float32))
    o = jnp.concatenate(outs, axis=1)
    res = jnp.dot(o, woutT_ref[:], preferred_element_type=jnp.float32) + bout_ref[:]
    out16 = res.reshape(WIN, WIN, D)
    zc = jnp.zeros((WIN, 8, D), jnp.float32)
    placed = jnp.concatenate([out16, zc], axis=1)
    hi_placed = jnp.concatenate([zc, out16], axis=1)
    placed = jnp.where(off == 8, hi_placed, placed)
    for k in (1, 2, 3):
        z1 = jnp.zeros((WIN, 2 * k, D), jnp.float32)
        z2 = jnp.zeros((WIN, 8 - 2 * k, D), jnp.float32)
        cand = jnp.concatenate([z1, out16, z2], axis=1)
        placed = jnp.where(off == 2 * k, cand, placed)
    out_ref[pl.ds(sy, WIN), pl.ds(sx8, 24), :] += placed

    @pl.when(t == KEEP - 1)
    def _fin():
        ir = jax.lax.broadcasted_iota(jnp.int32, (H, H), 0)
        ic = jax.lax.broadcasted_iota(jnp.int32, (H, H), 1)
        cnt = jnp.zeros((H, H), jnp.float32)
        for j in range(KEEP):
            yj = coords_ref[j, 0]
            xj = coords_ref[j, 1]
            inside = ((ir >= yj) & (ir < yj + WIN) &
                      (ic >= xj) & (ic < xj + WIN))
            cnt = cnt + inside.astype(jnp.float32)
        out_ref[:] = x_ref[:] + out_ref[:] / (cnt + 1e-10)[:, :, None]


def kernel(x, prob, W_fix, W_qkv, W_out, b_out):
    b, c, h, w = prob.shape
    x4 = x.reshape(b, H, H, D)
    call = pl.pallas_call(
        _body,
        grid=(KEEP,),
        in_specs=[
            pl.BlockSpec((c, h, w), lambda t: (0, 0, 0)),
            pl.BlockSpec((H, H, D), lambda t: (0, 0, 0)),
            pl.BlockSpec(memory_space=pltpu.SMEM),
            pl.BlockSpec((WIN * WIN, WIN * WIN), lambda t: (0, 0)),
            pl.BlockSpec((D, 3 * INNER), lambda t: (0, 0)),
            pl.BlockSpec((INNER, D), lambda t: (0, 0)),
            pl.BlockSpec((1, D), lambda t: (0, 0)),
        ],
        out_specs=pl.BlockSpec((H, H, D), lambda t: (0, 0, 0)),
        out_shape=jax.ShapeDtypeStruct((H, H, D), jnp.float32),
        scratch_shapes=[
            pltpu.SMEM((KEEP, 2), jnp.int32),
        ],
    )
    wfix = W_fix.reshape(-1)
    A = jnp.asarray(_A)
    wqkvT = W_qkv.T
    woutT = W_out.T
    bout = b_out.reshape(1, D)
    outs = [call(prob[i], x4[i], wfix, A, wqkvT, woutT, bout)
            for i in range(b)]
    return jnp.stack(outs, axis=0).reshape(b, H * H, D)


# speedup vs baseline: 51.2422x; 1.0779x over previous
import numpy as np
import jax
import jax.numpy as jnp
from jax.experimental import pallas as pl
from jax.experimental.pallas import tpu as pltpu

WIN = 16
HEADS = 8
DIM_HEAD = 64
ATT_SCALE = DIM_HEAD ** -0.5
GRID = 57
KEEP = 44
PAD = 4
H = 128
D = 192
INNER = HEADS * DIM_HEAD


def _interp_matrix():
    M = np.zeros((WIN, WIN), dtype=np.float32)
    scale = 15.0 / WIN
    for i in range(WIN):
        y = (i + 0.5) * scale
        p0 = int(np.floor(y))
        ly = y - p0
        M[i, p0] += 1.0 - ly
        M[i, p0 + 1] += ly
    return M


_A = np.kron(_interp_matrix(), _interp_matrix())


def _nms_offsets():
    offs = []
    for gdy in range(-PAD, PAD + 1):
        for gdx in range(-PAD, PAD + 1):
            if gdy == 0 and gdx == 0:
                continue
            dx, dy = 2.0 * gdx, 2.0 * gdy
            iw = max(0.0, min(15.0, dx + 15.0) - max(0.0, dx))
            ih = max(0.0, min(15.0, dy + 15.0) - max(0.0, dy))
            inter = iw * ih
            if inter / (450.0 - inter) > 0.2:
                tb = (gdy < 0) or (gdy == 0 and gdx < 0)
                offs.append((gdy, gdx, tb))
    return offs


_OFFS = _nms_offsets()


def _body(prob_ref, x_hbm, wfix_ref, A_ref, wqkvT_ref, woutT_ref, bout_ref,
          out_ref, xv_ref, coords_ref, dma_sem):
    i = pl.program_id(0)
    t = pl.program_id(1)

    @pl.when(t == 0)
    def _select():
        pltpu.make_async_copy(x_hbm.at[i], xv_ref, dma_sem).start()
        out_ref[0] = jnp.zeros((H, H, D), jnp.float32)

        p = prob_ref[0]
        ent = -jnp.sum(p * jnp.log2(p + 1e-10), axis=0)
        ws = jnp.zeros((GRID, GRID), jnp.float32)
        for u in range(8):
            for v in range(8):
                ws = ws + wfix_ref[u * 8 + v] * jax.lax.slice(
                    ent, (u, v), (u + GRID, v + GRID))
        ws = ws / 64.0

        neg = jnp.float32(-3.4e38)

        def pad2d(a, fill, dtype):
            side = jnp.full((GRID, PAD), fill, dtype)
            mid = jnp.concatenate([side, a, side], axis=1)
            top = jnp.full((PAD, GRID + 2 * PAD), fill, dtype)
            return jnp.concatenate([top, mid, top], axis=0)

        sp = pad2d(ws, neg, jnp.float32)
        higher = []
        for (dy, dx, tb) in _OFFS:
            snb = jax.lax.slice(
                sp, (PAD + dy, PAD + dx), (PAD + dy + GRID, PAD + dx + GRID))
            hi = snb > ws
            if tb:
                hi = hi | (snb == ws)
            higher.append(hi.astype(jnp.float32))

        def one_round(kf):
            kp = pad2d(kf, jnp.float32(0), jnp.float32)
            threat = jnp.zeros((GRID, GRID), jnp.float32)
            for (dy, dx, _), hi in zip(_OFFS, higher):
                knb = jax.lax.slice(
                    kp, (PAD + dy, PAD + dx), (PAD + dy + GRID, PAD + dx + GRID))
                threat = jnp.maximum(threat, knb * hi)
            return 1.0 - threat

        def cond(c):
            return c[1]

        def bd(c):
            kf, _ = c
            nk = one_round(kf)
            return nk, jnp.any(nk != kf)

        keptf, _ = jax.lax.while_loop(
            cond, bd, (jnp.ones((GRID, GRID), jnp.float32), jnp.bool_(True)))
        kept = keptf > 0.5

        iy = jax.lax.broadcasted_iota(jnp.int32, (GRID, GRID), 0)
        ix = jax.lax.broadcasted_iota(jnp.int32, (GRID, GRID), 1)
        lin = iy * GRID + ix
        big = jnp.int32(10 ** 8)

        def pick(j, takenf):
            avail = keptf * (1.0 - takenf)
            sc = jnp.where(avail > 0.5, ws, neg)
            m = jnp.max(sc)
            sl = jnp.min(jnp.where(sc == m, lin, big))
            coords_ref[j, 0] = (sl // GRID) * 2
            coords_ref[j, 1] = (sl % GRID) * 2
            return jnp.maximum(takenf, (lin == sl).astype(jnp.float32))

        jax.lax.fori_loop(0, KEEP, pick, jnp.zeros((GRID, GRID), jnp.float32))
        pltpu.make_async_copy(x_hbm.at[i], xv_ref, dma_sem).wait()

    sy = coords_ref[t, 0]
    sx = coords_ref[t, 1]
    sx8 = jnp.minimum((sx // 8) * 8, H - 24)
    off = sx - sx8
    wide = xv_ref[pl.ds(sy, WIN), pl.ds(sx8, 24), :]
    patch = jnp.where(off == 0, wide[:, 0:WIN, :], wide[:, 8:8 + WIN, :])
    for k in (1, 2, 3):
        patch = jnp.where(off == 2 * k, wide[:, 2 * k:2 * k + WIN, :], patch)
    patch = patch.reshape(WIN * WIN, D)
    xi = jnp.dot(A_ref[:], patch, preferred_element_type=jnp.float32)
    qkv = jnp.dot(xi, wqkvT_ref[:], preferred_element_type=jnp.float32)
    outs = []
    for hh in range(HEADS):
        qh = qkv[:, hh * DIM_HEAD:(hh + 1) * DIM_HEAD]
        kh = qkv[:, INNER + hh * DIM_HEAD:INNER + (hh + 1) * DIM_HEAD]
        vh = qkv[:, 2 * INNER + hh * DIM_HEAD:2 * INNER + (hh + 1) * DIM_HEAD]
        dots = jax.lax.dot_general(
            qh, kh, (((1,), (1,)), ((), ())),
            preferred_element_type=jnp.float32) * ATT_SCALE
        mx = jnp.max(dots, axis=1, keepdims=True)
        e = jnp.exp(dots - mx)
        a = e / jnp.sum(e, axis=1, keepdims=True)
        outs.append(jnp.dot(a, vh, preferred_element_type=jnp.float32))
    o = jnp.concatenate(outs, axis=1)
    res = jnp.dot(o, woutT_ref[:], preferred_element_type=jnp.float32) + bout_ref[:]
    out16 = res.reshape(WIN, WIN, D)
    zc = jnp.zeros((WIN, 8, D), jnp.float32)
    placed = jnp.concatenate([out16, zc], axis=1)
    hi_placed = jnp.concatenate([zc, out16], axis=1)
    placed = jnp.where(off == 8, hi_placed, placed)
    for k in (1, 2, 3):
        z1 = jnp.zeros((WIN, 2 * k, D), jnp.float32)
        z2 = jnp.zeros((WIN, 8 - 2 * k, D), jnp.float32)
        cand = jnp.concatenate([z1, out16, z2], axis=1)
        placed = jnp.where(off == 2 * k, cand, placed)
    out_ref[0, pl.ds(sy, WIN), pl.ds(sx8, 24), :] += placed

    @pl.when(t == KEEP - 1)
    def _fin():
        ir = jax.lax.broadcasted_iota(jnp.int32, (H, H), 0)
        ic = jax.lax.broadcasted_iota(jnp.int32, (H, H), 1)
        cnt = jnp.zeros((H, H), jnp.float32)
        for j in range(KEEP):
            yj = coords_ref[j, 0]
            xj = coords_ref[j, 1]
            inside = ((ir >= yj) & (ir < yj + WIN) &
                      (ic >= xj) & (ic < xj + WIN))
            cnt = cnt + inside.astype(jnp.float32)
        out_ref[0] = xv_ref[:] + out_ref[0] / (cnt + 1e-10)[:, :, None]


def kernel(x, prob, W_fix, W_qkv, W_out, b_out):
    b, c, h, w = prob.shape
    x4 = x.reshape(b, H, H, D)
    out = pl.pallas_call(
        _body,
        grid=(b, KEEP),
        in_specs=[
            pl.BlockSpec((1, c, h, w), lambda i, t: (i, 0, 0, 0)),
            pl.BlockSpec(memory_space=pl.ANY),
            pl.BlockSpec(memory_space=pltpu.SMEM),
            pl.BlockSpec((WIN * WIN, WIN * WIN), lambda i, t: (0, 0)),
            pl.BlockSpec((D, 3 * INNER), lambda i, t: (0, 0)),
            pl.BlockSpec((INNER, D), lambda i, t: (0, 0)),
            pl.BlockSpec((1, D), lambda i, t: (0, 0)),
        ],
        out_specs=pl.BlockSpec((1, H, H, D), lambda i, t: (i, 0, 0, 0)),
        out_shape=jax.ShapeDtypeStruct((b, H, H, D), jnp.float32),
        scratch_shapes=[
            pltpu.VMEM((H, H, D), jnp.float32),
            pltpu.SMEM((KEEP, 2), jnp.int32),
            pltpu.SemaphoreType.DMA,
        ],
    )(prob, x4, W_fix.reshape(-1), jnp.asarray(_A), W_qkv.T, W_out.T,
      b_out.reshape(1, D))
    return out.reshape(b, H * H, D)
